# baseline (speedup 1.0000x reference)
import functools

import jax
import jax.numpy as jnp
from jax.experimental import pallas as pl
from jax.experimental.pallas import tpu as pltpu

N = 100000
D = 768
A = 64
H = 256
K = 16

BR = 2048
NB = 49
RPAD = (NB * BR) // 128
NEG = -1e30


def _logits_kernel(x_ref, w1_ref, b1_ref, w2_ref, out_ref):
    i = pl.program_id(0)
    a = jnp.tanh(
        jnp.dot(x_ref[...], w1_ref[...], preferred_element_type=jnp.float32)
        + b1_ref[...]
    )
    s = jnp.dot(a, w2_ref[...], preferred_element_type=jnp.float32)
    s2 = s.reshape(BR // 128, 128)
    flat = (
        i * BR
        + jax.lax.broadcasted_iota(jnp.int32, (BR // 128, 128), 0) * 128
        + jax.lax.broadcasted_iota(jnp.int32, (BR // 128, 128), 1)
    )
    out_ref[...] = jnp.where(flat < N, s2, NEG)


def _finish_kernel(l_in_ref, x_hbm, wc1_ref, bc1_ref, wc2_ref, bc2_ref,
                   logit_ref, emb_ref, fw_ref, idx_ref,
                   l_ref, xk_ref, sem):
    l_ref[...] = l_in_ref[...]

    lane = jax.lax.broadcasted_iota(jnp.int32, (1, 128), 1)
    row_iota = jax.lax.iota(jnp.int32, RPAD)
    rm = jnp.max(l_ref[...], axis=1)
    rows = []
    cols = []
    vals = []
    copies = []
    for j in range(K):
        r = jnp.argmax(rm).astype(jnp.int32)
        rowv = l_ref[pl.ds(r, 1), :]
        c = jnp.argmax(rowv[0, :]).astype(jnp.int32)
        v = jnp.max(rowv)
        flat_idx = r * 128 + c
        rows.append(r)
        cols.append(c)
        vals.append(v)
        newrow = jnp.where(lane == c, NEG, rowv)
        l_ref[pl.ds(r, 1), :] = newrow
        rm = jnp.where(row_iota == r, jnp.max(newrow), rm)
        cp = pltpu.make_async_copy(
            x_hbm.at[pl.ds(flat_idx, 1), :], xk_ref.at[pl.ds(j, 1), :], sem)
        cp.start()
        copies.append(cp)

    exps = [jnp.exp(v - vals[0]) for v in vals]
    denom = exps[0]
    for e in exps[1:]:
        denom = denom + e
    ws = [e / denom for e in exps]

    lane_k = jax.lax.broadcasted_iota(jnp.int32, (1, K), 1)
    wrow = jnp.zeros((1, K), jnp.float32)
    irow = jnp.zeros((1, K), jnp.int32)
    for j in range(K):
        wrow = jnp.where(lane_k == j, ws[j], wrow)
        irow = jnp.where(lane_k == j, rows[j] * 128 + cols[j], irow)
    idx_ref[...] = irow

    for cp in copies:
        cp.wait()

    emb = jnp.dot(wrow, xk_ref[...], preferred_element_type=jnp.float32)
    emb_ref[...] = emb
    h = jnp.maximum(
        jnp.dot(emb, wc1_ref[...], preferred_element_type=jnp.float32)
        + bc1_ref[...], 0.0)
    logit_ref[...] = (
        jnp.dot(h, wc2_ref[...], preferred_element_type=jnp.float32)
        + bc2_ref[...])

    fw_ref[...] = jnp.zeros((RPAD, 128), jnp.float32)
    for j in range(K):
        cur = fw_ref[pl.ds(rows[j], 1), :]
        fw_ref[pl.ds(rows[j], 1), :] = jnp.where(lane == cols[j], ws[j], cur)


@jax.jit
def kernel(x, W1, b1, W2, b2, Wc1, bc1, Wc2, bc2):
    logits = pl.pallas_call(
        _logits_kernel,
        grid=(NB,),
        in_specs=[
            pl.BlockSpec((BR, D), lambda i: (i, 0)),
            pl.BlockSpec((D, A), lambda i: (0, 0)),
            pl.BlockSpec((1, A), lambda i: (0, 0)),
            pl.BlockSpec((A, 1), lambda i: (0, 0)),
        ],
        out_specs=pl.BlockSpec((BR // 128, 128), lambda i: (i, 0)),
        out_shape=jax.ShapeDtypeStruct((RPAD, 128), jnp.float32),
        compiler_params=pltpu.CompilerParams(
            dimension_semantics=("parallel",)),
    )(x, W1, b1.reshape(1, A), W2)

    logit2d, emb2d, fw2d, idx2d = pl.pallas_call(
        _finish_kernel,
        in_specs=[
            pl.BlockSpec(memory_space=pltpu.MemorySpace.VMEM),
            pl.BlockSpec(memory_space=pl.ANY),
            pl.BlockSpec(memory_space=pltpu.MemorySpace.VMEM),
            pl.BlockSpec(memory_space=pltpu.MemorySpace.VMEM),
            pl.BlockSpec(memory_space=pltpu.MemorySpace.VMEM),
            pl.BlockSpec(memory_space=pltpu.MemorySpace.VMEM),
        ],
        out_shape=(
            jax.ShapeDtypeStruct((1, 1), jnp.float32),
            jax.ShapeDtypeStruct((1, D), jnp.float32),
            jax.ShapeDtypeStruct((RPAD, 128), jnp.float32),
            jax.ShapeDtypeStruct((1, K), jnp.int32),
        ),
        scratch_shapes=[
            pltpu.VMEM((RPAD, 128), jnp.float32),
            pltpu.VMEM((K, D), jnp.float32),
            pltpu.SemaphoreType.DMA,
        ],
    )(logits, x, Wc1, bc1.reshape(1, H), Wc2, bc2.reshape(1, 1))

    logit = logit2d.reshape(())
    slide_embedding = emb2d.reshape(D)
    full_weights = fw2d.reshape(RPAD * 128)[:N]
    topk_idx = idx2d.reshape(K)
    return (logit, slide_embedding, full_weights, topk_idx)


# speedup vs baseline: 1.0889x; 1.0889x over previous
import functools

import jax
import jax.numpy as jnp
from jax.experimental import pallas as pl
from jax.experimental.pallas import tpu as pltpu

N = 100000
D = 768
A = 64
H = 256
K = 16

BR = 4096
NB = 25
RPAD = (NB * BR) // 128
NEG = -1e30


def _logits_kernel(x_ref, w1_ref, b1_ref, w2_ref, out_ref):
    i = pl.program_id(0)
    a = jnp.tanh(
        jnp.dot(x_ref[...], w1_ref[...], preferred_element_type=jnp.float32)
        + b1_ref[...]
    )
    s = jnp.dot(a, w2_ref[...], preferred_element_type=jnp.float32)
    s2 = s.reshape(BR // 128, 128)
    flat = (
        i * BR
        + jax.lax.broadcasted_iota(jnp.int32, (BR // 128, 128), 0) * 128
        + jax.lax.broadcasted_iota(jnp.int32, (BR // 128, 128), 1)
    )
    out_ref[...] = jnp.where(flat < N, s2, NEG)


def _finish_kernel(l_in_ref, x_hbm, wc1_ref, bc1_ref, wc2_ref, bc2_ref,
                   logit_ref, emb_ref, fw_ref, idx_ref,
                   l_ref, xk_ref, sem):
    l_ref[...] = l_in_ref[...]

    lane = jax.lax.broadcasted_iota(jnp.int32, (1, 128), 1)
    row_iota = jax.lax.iota(jnp.int32, RPAD)
    rm = jnp.max(l_ref[...], axis=1)
    rows = []
    cols = []
    vals = []
    copies = []
    for j in range(K):
        r = jnp.argmax(rm).astype(jnp.int32)
        rowv = l_ref[pl.ds(r, 1), :]
        c = jnp.argmax(rowv[0, :]).astype(jnp.int32)
        v = jnp.max(rowv)
        flat_idx = r * 128 + c
        rows.append(r)
        cols.append(c)
        vals.append(v)
        newrow = jnp.where(lane == c, NEG, rowv)
        l_ref[pl.ds(r, 1), :] = newrow
        rm = jnp.where(row_iota == r, jnp.max(newrow), rm)
        cp = pltpu.make_async_copy(
            x_hbm.at[pl.ds(flat_idx, 1), :], xk_ref.at[pl.ds(j, 1), :], sem)
        cp.start()
        copies.append(cp)

    exps = [jnp.exp(v - vals[0]) for v in vals]
    denom = exps[0]
    for e in exps[1:]:
        denom = denom + e
    ws = [e / denom for e in exps]

    lane_k = jax.lax.broadcasted_iota(jnp.int32, (1, K), 1)
    wrow = jnp.zeros((1, K), jnp.float32)
    irow = jnp.zeros((1, K), jnp.int32)
    for j in range(K):
        wrow = jnp.where(lane_k == j, ws[j], wrow)
        irow = jnp.where(lane_k == j, rows[j] * 128 + cols[j], irow)
    idx_ref[...] = irow

    for cp in copies:
        cp.wait()

    emb = jnp.dot(wrow, xk_ref[...], preferred_element_type=jnp.float32)
    emb_ref[...] = emb
    h = jnp.maximum(
        jnp.dot(emb, wc1_ref[...], preferred_element_type=jnp.float32)
        + bc1_ref[...], 0.0)
    logit_ref[...] = (
        jnp.dot(h, wc2_ref[...], preferred_element_type=jnp.float32)
        + bc2_ref[...])

    fw_ref[...] = jnp.zeros((RPAD, 128), jnp.float32)
    for j in range(K):
        cur = fw_ref[pl.ds(rows[j], 1), :]
        fw_ref[pl.ds(rows[j], 1), :] = jnp.where(lane == cols[j], ws[j], cur)


@jax.jit
def kernel(x, W1, b1, W2, b2, Wc1, bc1, Wc2, bc2):
    logits = pl.pallas_call(
        _logits_kernel,
        grid=(NB,),
        in_specs=[
            pl.BlockSpec((BR, D), lambda i: (i, 0)),
            pl.BlockSpec((D, A), lambda i: (0, 0)),
            pl.BlockSpec((1, A), lambda i: (0, 0)),
            pl.BlockSpec((A, 1), lambda i: (0, 0)),
        ],
        out_specs=pl.BlockSpec((BR // 128, 128), lambda i: (i, 0)),
        out_shape=jax.ShapeDtypeStruct((RPAD, 128), jnp.float32),
        compiler_params=pltpu.CompilerParams(
            dimension_semantics=("parallel",)),
    )(x, W1, b1.reshape(1, A), W2)

    logit2d, emb2d, fw2d, idx2d = pl.pallas_call(
        _finish_kernel,
        in_specs=[
            pl.BlockSpec(memory_space=pltpu.MemorySpace.VMEM),
            pl.BlockSpec(memory_space=pl.ANY),
            pl.BlockSpec(memory_space=pltpu.MemorySpace.VMEM),
            pl.BlockSpec(memory_space=pltpu.MemorySpace.VMEM),
            pl.BlockSpec(memory_space=pltpu.MemorySpace.VMEM),
            pl.BlockSpec(memory_space=pltpu.MemorySpace.VMEM),
        ],
        out_shape=(
            jax.ShapeDtypeStruct((1, 1), jnp.float32),
            jax.ShapeDtypeStruct((1, D), jnp.float32),
            jax.ShapeDtypeStruct((RPAD, 128), jnp.float32),
            jax.ShapeDtypeStruct((1, K), jnp.int32),
        ),
        scratch_shapes=[
            pltpu.VMEM((RPAD, 128), jnp.float32),
            pltpu.VMEM((K, D), jnp.float32),
            pltpu.SemaphoreType.DMA,
        ],
    )(logits, x, Wc1, bc1.reshape(1, H), Wc2, bc2.reshape(1, 1))

    logit = logit2d.reshape(())
    slide_embedding = emb2d.reshape(D)
    full_weights = fw2d.reshape(RPAD * 128)[:N]
    topk_idx = idx2d.reshape(K)
    return (logit, slide_embedding, full_weights, topk_idx)


# speedup vs baseline: 1.1006x; 1.0107x over previous
import jax
import jax.numpy as jnp
from jax.experimental import pallas as pl
from jax.experimental.pallas import tpu as pltpu

N = 100000
D = 768
A = 64
H = 256
K = 16

BR = 4096
NB = 25
SR = BR // 128
RPAD = NB * SR
NEG = -1e30


def _fused_kernel(x_ref, w1_ref, b1_ref, w2_ref, x_hbm,
                  wc1_ref, bc1_ref, wc2_ref, bc2_ref,
                  logit_ref, emb_ref, fw_ref, idx_ref,
                  l_ref, xk_ref, sem):
    i = pl.program_id(0)
    a = jnp.tanh(
        jnp.dot(x_ref[...], w1_ref[...], preferred_element_type=jnp.float32)
        + b1_ref[...]
    )
    s = jnp.dot(a, w2_ref[...], preferred_element_type=jnp.float32)
    s2 = s.reshape(SR, 128)
    flat = (
        i * BR
        + jax.lax.broadcasted_iota(jnp.int32, (SR, 128), 0) * 128
        + jax.lax.broadcasted_iota(jnp.int32, (SR, 128), 1)
    )
    l_ref[pl.ds(i * SR, SR), :] = jnp.where(flat < N, s2, NEG)

    @pl.when(i == NB - 1)
    def _finish():
        lane = jax.lax.broadcasted_iota(jnp.int32, (1, 128), 1)
        row_iota = jax.lax.iota(jnp.int32, RPAD)
        rm = jnp.max(l_ref[...], axis=1)
        rows = []
        cols = []
        vals = []
        copies = []
        for j in range(K):
            r = jnp.argmax(rm).astype(jnp.int32)
            rowv = l_ref[pl.ds(r, 1), :]
            c = jnp.argmax(rowv[0, :]).astype(jnp.int32)
            v = jnp.max(rowv)
            flat_idx = r * 128 + c
            rows.append(r)
            cols.append(c)
            vals.append(v)
            newrow = jnp.where(lane == c, NEG, rowv)
            l_ref[pl.ds(r, 1), :] = newrow
            rm = jnp.where(row_iota == r, jnp.max(newrow), rm)
            cp = pltpu.make_async_copy(
                x_hbm.at[pl.ds(flat_idx, 1), :], xk_ref.at[pl.ds(j, 1), :],
                sem)
            cp.start()
            copies.append(cp)

        exps = [jnp.exp(v - vals[0]) for v in vals]
        denom = exps[0]
        for e in exps[1:]:
            denom = denom + e
        ws = [e / denom for e in exps]

        lane_k = jax.lax.broadcasted_iota(jnp.int32, (1, K), 1)
        wrow = jnp.zeros((1, K), jnp.float32)
        irow = jnp.zeros((1, K), jnp.int32)
        for j in range(K):
            wrow = jnp.where(lane_k == j, ws[j], wrow)
            irow = jnp.where(lane_k == j, rows[j] * 128 + cols[j], irow)
        idx_ref[...] = irow

        for cp in copies:
            cp.wait()

        emb = jnp.dot(wrow, xk_ref[...],
                      preferred_element_type=jnp.float32)
        emb_ref[...] = emb
        h = jnp.maximum(
            jnp.dot(emb, wc1_ref[...], preferred_element_type=jnp.float32)
            + bc1_ref[...], 0.0)
        logit_ref[...] = (
            jnp.dot(h, wc2_ref[...], preferred_element_type=jnp.float32)
            + bc2_ref[...])

        fw_ref[...] = jnp.zeros((RPAD, 128), jnp.float32)
        for j in range(K):
            cur = fw_ref[pl.ds(rows[j], 1), :]
            fw_ref[pl.ds(rows[j], 1), :] = jnp.where(lane == cols[j],
                                                     ws[j], cur)


@jax.jit
def kernel(x, W1, b1, W2, b2, Wc1, bc1, Wc2, bc2):
    logit2d, emb2d, fw2d, idx2d = pl.pallas_call(
        _fused_kernel,
        grid=(NB,),
        in_specs=[
            pl.BlockSpec((BR, D), lambda i: (i, 0)),
            pl.BlockSpec((D, A), lambda i: (0, 0)),
            pl.BlockSpec((1, A), lambda i: (0, 0)),
            pl.BlockSpec((A, 1), lambda i: (0, 0)),
            pl.BlockSpec(memory_space=pl.ANY),
            pl.BlockSpec((D, H), lambda i: (0, 0)),
            pl.BlockSpec((1, H), lambda i: (0, 0)),
            pl.BlockSpec((H, 1), lambda i: (0, 0)),
            pl.BlockSpec((1, 1), lambda i: (0, 0)),
        ],
        out_specs=(
            pl.BlockSpec((1, 1), lambda i: (0, 0)),
            pl.BlockSpec((1, D), lambda i: (0, 0)),
            pl.BlockSpec((RPAD, 128), lambda i: (0, 0)),
            pl.BlockSpec((1, K), lambda i: (0, 0)),
        ),
        out_shape=(
            jax.ShapeDtypeStruct((1, 1), jnp.float32),
            jax.ShapeDtypeStruct((1, D), jnp.float32),
            jax.ShapeDtypeStruct((RPAD, 128), jnp.float32),
            jax.ShapeDtypeStruct((1, K), jnp.int32),
        ),
        scratch_shapes=[
            pltpu.VMEM((RPAD, 128), jnp.float32),
            pltpu.VMEM((K, D), jnp.float32),
            pltpu.SemaphoreType.DMA,
        ],
        compiler_params=pltpu.CompilerParams(
            dimension_semantics=("arbitrary",)),
    )(x, W1, b1.reshape(1, A), W2, x,
      Wc1, bc1.reshape(1, H), Wc2, bc2.reshape(1, 1))

    logit = logit2d.reshape(())
    slide_embedding = emb2d.reshape(D)
    full_weights = fw2d.reshape(RPAD * 128)[:N]
    topk_idx = idx2d.reshape(K)
    return (logit, slide_embedding, full_weights, topk_idx)
